# Initial kernel scaffold; baseline (speedup 1.0000x reference)
import jax
import jax.numpy as jnp
from jax.experimental import pallas as pl


def kernel(images, patch_w, patch_b, cls_token, pos_embed, router_wq, router_bq, router_wk, router_bk, pos_bias, qkv_w, qkv_b, proj_w, proj_b, ln1_g, ln1_b, ln2_g, ln2_b, mlp_w1, mlp_b1, mlp_w2, mlp_b2, lnf_g, lnf_b):
    raise NotImplementedError("write your pallas kernel here")


# speedup vs baseline: 4.2685x; 4.2685x over previous
import jax
import jax.numpy as jnp
from jax.experimental import pallas as pl
from functools import partial

L = 4
D = 768
H = 12
HD = 64
P = 256
G = 16
K = 8
PS = 14
TEMP = 0.1
MLP_D = 3072
SCALE = HD ** -0.5
NEG = -1e9


def _ln_rows(x, g, b):
    m = jnp.mean(x, axis=-1, keepdims=True)
    v = jnp.mean((x - m) ** 2, axis=-1, keepdims=True)
    return (x - m) * jax.lax.rsqrt(v + 1e-5) * g + b


def _l2n(x):
    n2 = jnp.sum(x * x, axis=-1, keepdims=True)
    n = jnp.sqrt(n2)
    return x / jnp.maximum(n, 1e-12)


def _mm(a, b):
    return jnp.dot(a, b, preferred_element_type=jnp.float32)


def _mmT(a, b):
    return jax.lax.dot_general(a, b, (((1,), (1,)), ((), ())),
                               preferred_element_type=jnp.float32)


def _embed_kernel(xp_ref, w_ref, b_ref, pos_ref, out_ref):
    x = xp_ref[0]
    out_ref[0] = _mm(x, w_ref[...]) + b_ref[...] + pos_ref[...]


def _layer_kernel(tokp_ref, tokc_ref, wq_ref, bq_ref, wk_ref, bk_ref, bias_ref,
                  qkvw_ref, qkvb_ref, projw_ref, projb_ref,
                  g1_ref, be1_ref, g2_ref, be2_ref,
                  w1_ref, mb1_ref, w2_ref, mb2_ref,
                  outp_ref, outc_ref):
    tokp = tokp_ref[0]
    tokc = tokc_ref[0]
    g1 = g1_ref[...]
    be1 = be1_ref[...]
    xn_p = _ln_rows(tokp, g1, be1)
    xn_c = _ln_rows(tokc, g1, be1)

    q = _l2n(_mm(xn_p, wq_ref[...]) + bq_ref[...])
    k = _l2n(_mm(xn_p, wk_ref[...]) + bk_ref[...])
    iota_q = jax.lax.broadcasted_iota(jnp.int32, (P, P), 1)
    iota_p = jax.lax.broadcasted_iota(jnp.int32, (P, P), 0)
    sc = _mmT(q, k) + bias_ref[...]
    sc = jnp.where(iota_q == iota_p, NEG, sc)

    work = sc
    vals = []
    idxs = []
    for _ in range(K):
        m = jnp.max(work, axis=-1, keepdims=True)
        idx = jnp.min(jnp.where(work == m, iota_q, P),
                      axis=-1, keepdims=True)
        work = jnp.where(iota_q == idx, NEG, work)
        vals.append(m / TEMP)
        idxs.append(idx)

    vmax = vals[0]
    for t in range(1, K):
        vmax = jnp.maximum(vmax, vals[t])
    es = [jnp.exp(v - vmax) for v in vals]
    den = es[0]
    for t in range(1, K):
        den = den + es[t]
    rw_dense = jnp.zeros((P, P), jnp.float32)
    for t in range(K):
        rw_dense = rw_dense + jnp.where(iota_q == idxs[t], es[t] / den, 0.0)
    routed = rw_dense > 0.0

    qkvb = qkvb_ref[...]
    qkv_p = _mm(xn_p, qkvw_ref[...]) + qkvb
    qkv_c = _mm(xn_c, qkvw_ref[...]) + qkvb

    oc_parts = []
    op_parts = []
    for h in range(H):
        q0 = h * HD
        Qh = qkv_p[:, q0:q0 + HD]
        Kh = qkv_p[:, D + q0:D + q0 + HD]
        Vh = qkv_p[:, 2 * D + q0:2 * D + q0 + HD]
        qc = qkv_c[:, q0:q0 + HD]
        kc = qkv_c[:, D + q0:D + q0 + HD]
        vc = qkv_c[:, 2 * D + q0:2 * D + q0 + HD]

        lp = _mmT(qc, Kh) * SCALE
        ls = jnp.sum(qc * kc, axis=-1, keepdims=True) * SCALE
        mx = jnp.maximum(jnp.max(lp, axis=-1, keepdims=True), ls)
        ep = jnp.exp(lp - mx)
        es_c = jnp.exp(ls - mx)
        denom_c = es_c + jnp.sum(ep, axis=-1, keepdims=True)
        oc_h = (es_c * vc + _mm(ep, Vh)) / denom_c
        oc_parts.append(oc_h)

        Z = _mmT(Qh, Kh) * SCALE
        Zm = jnp.where(routed, Z, NEG)
        zmax = jnp.max(Zm, axis=-1, keepdims=True)
        A = jnp.where(routed, jnp.exp(Zm - zmax), 0.0)
        A = A / jnp.sum(A, axis=-1, keepdims=True)
        W = A * rw_dense
        W = W / (jnp.sum(W, axis=-1, keepdims=True) + 1e-6)
        op_parts.append(_mm(W, Vh))

    oc = jnp.concatenate(oc_parts, axis=-1)
    op = jnp.concatenate(op_parts, axis=-1)

    projb = projb_ref[...]
    tokp1 = tokp + _mm(op, projw_ref[...]) + projb
    tokc1 = tokc + _mm(oc, projw_ref[...]) + projb

    g2 = g2_ref[...]
    be2 = be2_ref[...]
    mb1 = mb1_ref[...]
    mb2 = mb2_ref[...]
    xn2_p = _ln_rows(tokp1, g2, be2)
    xn2_c = _ln_rows(tokc1, g2, be2)
    h_p = jax.nn.gelu(_mm(xn2_p, w1_ref[...]) + mb1)
    h_c = jax.nn.gelu(_mm(xn2_c, w1_ref[...]) + mb1)
    outp_ref[0] = tokp1 + _mm(h_p, w2_ref[...]) + mb2
    outc_ref[0] = tokc1 + _mm(h_c, w2_ref[...]) + mb2


def _final_kernel(tokc_ref, g_ref, b_ref, out_ref):
    out_ref[...] = _ln_rows(tokc_ref[:, 0, :], g_ref[...], b_ref[...])


def _full(shape):
    nd = len(shape)
    return pl.BlockSpec(shape, lambda b: (0,) * nd)


def kernel(images, patch_w, patch_b, cls_token, pos_embed, router_wq, router_bq,
           router_wk, router_bk, pos_bias, qkv_w, qkv_b, proj_w, proj_b,
           ln1_g, ln1_b, ln2_g, ln2_b, mlp_w1, mlp_b1, mlp_w2, mlp_b2,
           lnf_g, lnf_b, interpret=False):
    B = images.shape[0]
    CIN = 3 * PS * PS
    x = images.reshape(B, 3, G, PS, G, PS).transpose(0, 2, 4, 1, 3, 5)
    x = x.reshape(B, P, CIN)

    pos_p = pos_embed[0, 1:, :]
    tok_p = pl.pallas_call(
        _embed_kernel,
        grid=(B,),
        in_specs=[
            pl.BlockSpec((1, P, CIN), lambda b: (b, 0, 0)),
            _full((CIN, D)),
            _full((1, D)),
            _full((P, D)),
        ],
        out_specs=pl.BlockSpec((1, P, D), lambda b: (b, 0, 0)),
        out_shape=jax.ShapeDtypeStruct((B, P, D), jnp.float32),
        interpret=interpret,
    )(x, patch_w, patch_b.reshape(1, D), pos_p)

    tok_c = jnp.broadcast_to(cls_token[0] + pos_embed[0, :1, :], (B, 1, D))

    layer_call = pl.pallas_call(
        _layer_kernel,
        grid=(B,),
        in_specs=[
            pl.BlockSpec((1, P, D), lambda b: (b, 0, 0)),
            pl.BlockSpec((1, 1, D), lambda b: (b, 0, 0)),
            _full((D, D)), _full((1, D)),
            _full((D, D)), _full((1, D)),
            _full((P, P)),
            _full((D, 3 * D)), _full((1, 3 * D)),
            _full((D, D)), _full((1, D)),
            _full((1, D)), _full((1, D)),
            _full((1, D)), _full((1, D)),
            _full((D, MLP_D)), _full((1, MLP_D)),
            _full((MLP_D, D)), _full((1, D)),
        ],
        out_specs=[
            pl.BlockSpec((1, P, D), lambda b: (b, 0, 0)),
            pl.BlockSpec((1, 1, D), lambda b: (b, 0, 0)),
        ],
        out_shape=[
            jax.ShapeDtypeStruct((B, P, D), jnp.float32),
            jax.ShapeDtypeStruct((B, 1, D), jnp.float32),
        ],
        interpret=interpret,
    )

    for i in range(L):
        tok_p, tok_c = layer_call(
            tok_p, tok_c,
            router_wq[i], router_bq[i].reshape(1, D),
            router_wk[i], router_bk[i].reshape(1, D),
            pos_bias[i],
            qkv_w[i], qkv_b[i].reshape(1, 3 * D),
            proj_w[i], proj_b[i].reshape(1, D),
            ln1_g[i].reshape(1, D), ln1_b[i].reshape(1, D),
            ln2_g[i].reshape(1, D), ln2_b[i].reshape(1, D),
            mlp_w1[i], mlp_b1[i].reshape(1, MLP_D),
            mlp_w2[i], mlp_b2[i].reshape(1, D),
        )

    out = pl.pallas_call(
        _final_kernel,
        in_specs=[
            pl.BlockSpec((B, 1, D), lambda: (0, 0, 0)),
            pl.BlockSpec((1, D), lambda: (0, 0)),
            pl.BlockSpec((1, D), lambda: (0, 0)),
        ],
        out_specs=pl.BlockSpec((B, D), lambda: (0, 0)),
        out_shape=jax.ShapeDtypeStruct((B, D), jnp.float32),
        interpret=interpret,
    )(tok_c, lnf_g.reshape(1, D), lnf_b.reshape(1, D))
    return out
